# baseline (speedup 1.0000x reference)
import jax
import jax.numpy as jnp
from jax import lax
from jax.experimental import pallas as pl
from jax.experimental.pallas import tpu as pltpu
from jax.experimental.pallas import tpu_sc as plsc

N = 10000
NP = 10240
D = 256
DH = 128
E_TOT = 330000
CH = 112
NTILES = 16
NCHUNK = 186
E_PAD = NTILES * CH * NCHUNK
EPT = E_PAD // NTILES
RPT = NP // NTILES
CH1 = 496
N1 = 21
WSLAB = E_PAD // 32
EPS = 1e-16
ROW_BLK = 2048
GRID_R = NP // ROW_BLK


def _sc_alpha_body(src_hbm, dst_hbm, asrc_hbm, adst_hbm, ex_out,
                   asrc_v, adst_v, sidx_v, didx_v, ex_v):
    c = lax.axis_index("c")
    s = lax.axis_index("s")
    wid = s * 2 + c
    pltpu.sync_copy(asrc_hbm, asrc_v)
    pltpu.sync_copy(adst_hbm, adst_v)
    wbase = wid * WSLAB

    def chunk(ci, carry):
        base = wbase + ci * CH1
        pltpu.sync_copy(src_hbm.at[pl.ds(base, CH1)], sidx_v)
        pltpu.sync_copy(dst_hbm.at[pl.ds(base, CH1)], didx_v)

        def grp(j, _):
            s16 = sidx_v[pl.ds(j * 16, 16)]
            d16 = didx_v[pl.ds(j * 16, 16)]
            al = (plsc.load_gather(asrc_v, [s16])
                  + plsc.load_gather(adst_v, [d16]))
            al = jnp.where(al >= 0.0, al, al * jnp.float32(0.2))
            ex = jnp.exp(al)
            gi = base + j * 16 + lax.iota(jnp.int32, 16)
            ex = jnp.where(gi < E_TOT, ex, jnp.float32(0.0))
            ex_v[pl.ds(j * 16, 16)] = ex
            return 0

        lax.fori_loop(0, CH1 // 16, grp, 0)
        pltpu.sync_copy(ex_v, ex_out.at[pl.ds(base, CH1)])
        return carry

    lax.fori_loop(0, N1, chunk, 0)


_sc_alpha = pl.kernel(
    _sc_alpha_body,
    out_type=jax.ShapeDtypeStruct((E_PAD,), jnp.float32),
    mesh=plsc.VectorSubcoreMesh(core_axis_name="c", subcore_axis_name="s"),
    compiler_params=pltpu.CompilerParams(needs_layout_passes=False),
    scratch_types=[
        pltpu.VMEM((NP,), jnp.float32),
        pltpu.VMEM((NP,), jnp.float32),
        pltpu.VMEM((CH1,), jnp.int32),
        pltpu.VMEM((CH1,), jnp.int32),
        pltpu.VMEM((CH1,), jnp.float32),
    ],
)


def _sc_edge_body(src_hbm, dst_hbm, ex_hbm, h0_hbm, h1_hbm,
                  z2_hbm, z1_hbm,
                  s0_out, s1_out, dn_out,
                  sidx0, didx0, ex0, sidx1, didx1, ex1,
                  sidx2, didx2, ex2, sidx3, didx3, ex3,
                  sidx4, didx4, ex4, sidx5, didx5, ex5,
                  rows0, rows1, rows2,
                  s_sh, dn_sh,
                  gsem0, gsem1, gsem2, ssem0, ssem1, ssem2,
                  isem0, isem1, isem2, isem3, isem4, isem5):
    c = lax.axis_index("c")
    s = lax.axis_index("s")
    sidx = [sidx0, sidx1, sidx2, sidx3, sidx4, sidx5]
    didx = [didx0, didx1, didx2, didx3, didx4, didx5]
    exv = [ex0, ex1, ex2, ex3, ex4, ex5]
    rows = [rows0, rows1, rows2]
    gsem = [gsem0, gsem1, gsem2]
    ssem = [ssem0, ssem1, ssem2]
    isem = [isem0, isem1, isem2, isem3, isem4, isem5]

    r0 = s * RPT
    pltpu.sync_copy(z2_hbm.at[pl.ds(r0, RPT)], s_sh.at[pl.ds(r0, RPT)])
    pltpu.sync_copy(z1_hbm.at[pl.ds(r0, RPT)], dn_sh.at[pl.ds(r0, RPT)])
    plsc.subcore_barrier()

    tbase = s * EPT

    def idx_load(q, k):
        base = tbase + k * CH
        pltpu.async_copy(src_hbm.at[pl.ds(base, CH)], sidx[q], isem[q])
        pltpu.async_copy(dst_hbm.at[pl.ds(base, CH)], didx[q], isem[q])
        pltpu.async_copy(ex_hbm.at[pl.ds(base, CH)], exv[q], isem[q])

    def idx_wait(q, k):
        base = tbase + k * CH
        pltpu.make_async_copy(
            src_hbm.at[pl.ds(base, CH)], sidx[q], isem[q]).wait()
        pltpu.make_async_copy(
            dst_hbm.at[pl.ds(base, CH)], didx[q], isem[q]).wait()
        pltpu.make_async_copy(
            ex_hbm.at[pl.ds(base, CH)], exv[q], isem[q]).wait()

    def gather(q, b):
        @pl.when(c == 0)
        def _():
            pltpu.async_copy(h0_hbm.at[sidx[q]], rows[b], gsem[b])

        @pl.when(c == 1)
        def _():
            pltpu.async_copy(h1_hbm.at[sidx[q]], rows[b], gsem[b])

    def drain_scatter(q, b):
        pltpu.make_async_copy(rows[b], s_sh.at[didx[q]], ssem[b]).wait()
        pltpu.make_async_copy(exv[q], dn_sh.at[didx[q]], ssem[b]).wait()

    def slot(k, u):
        b = u % 3
        q = u % 6
        pltpu.make_async_copy(h0_hbm.at[sidx[q]], rows[b], gsem[b]).wait()

        def scale(g, _):
            ex16 = exv[q][pl.ds(g * 16, 16)]
            for j in range(16):
                sv = ex16[j]
                row = g * 16 + j
                for r in range(DH // 16):
                    sl = pl.ds(r * 16, 16)
                    rows[b][row, sl] = rows[b][row, sl] * sv
            return 0

        lax.fori_loop(0, CH // 16, scale, 0)
        pltpu.async_copy(rows[b], s_sh.at[didx[q]], ssem[b], add=True)
        pltpu.async_copy(exv[q], dn_sh.at[didx[q]], ssem[b], add=True)

        @pl.when(k >= 1)
        def _():
            drain_scatter((u - 1) % 6, (u - 1) % 3)

        @pl.when(k + 4 < NCHUNK)
        def _():
            idx_load((u + 4) % 6, k + 4)

        @pl.when(k + 2 < NCHUNK)
        def _():
            idx_wait((u + 2) % 6, k + 2)
            gather((u + 2) % 6, (u + 2) % 3)

    for q in range(4):
        idx_load(q, q)
    for j in range(2):
        idx_wait(j, j)
        gather(j, j)

    def group(g, carry):
        for u in range(6):
            slot(6 * g + u, u)
        return carry

    lax.fori_loop(0, NCHUNK // 6, group, 0)
    drain_scatter((NCHUNK - 1) % 6, (NCHUNK - 1) % 3)
    plsc.subcore_barrier()

    @pl.when(c == 0)
    def _():
        pltpu.sync_copy(s_sh.at[pl.ds(r0, RPT)], s0_out.at[pl.ds(r0, RPT)])
        pltpu.sync_copy(dn_sh.at[pl.ds(r0, RPT)], dn_out.at[pl.ds(r0, RPT)])

    @pl.when(c == 1)
    def _():
        pltpu.sync_copy(s_sh.at[pl.ds(r0, RPT)], s1_out.at[pl.ds(r0, RPT)])


_sc_edge = pl.kernel(
    _sc_edge_body,
    out_type=[
        jax.ShapeDtypeStruct((NP, DH), jnp.float32),
        jax.ShapeDtypeStruct((NP, DH), jnp.float32),
        jax.ShapeDtypeStruct((NP,), jnp.float32),
    ],
    mesh=plsc.VectorSubcoreMesh(core_axis_name="c", subcore_axis_name="s"),
    compiler_params=pltpu.CompilerParams(needs_layout_passes=False),
    scratch_types=[
        t for _ in range(6) for t in (
            pltpu.VMEM((CH,), jnp.int32),
            pltpu.VMEM((CH,), jnp.int32),
            pltpu.VMEM((CH,), jnp.float32),
        )
    ] + [pltpu.VMEM((CH, DH), jnp.float32)] * 3 + [
        pltpu.VMEM_SHARED((NP, DH), jnp.float32),
        pltpu.VMEM_SHARED((NP,), jnp.float32),
    ] + [pltpu.SemaphoreType.DMA] * 12,
)


def _proj_body(x_ref, wt_ref, b0_ref, b1_ref, o0_ref, o1_ref):
    h = jnp.dot(x_ref[...], wt_ref[...], preferred_element_type=jnp.float32)
    o0_ref[...] = h[:, :DH] + b0_ref[...]
    o1_ref[...] = h[:, DH:] + b1_ref[...]


def _proj(x, wt, b0, b1):
    return pl.pallas_call(
        _proj_body,
        grid=(GRID_R,),
        in_specs=[
            pl.BlockSpec((ROW_BLK, x.shape[1]), lambda i: (i, 0)),
            pl.BlockSpec(wt.shape, lambda i: (0, 0)),
            pl.BlockSpec((1, DH), lambda i: (0, 0)),
            pl.BlockSpec((1, DH), lambda i: (0, 0)),
        ],
        out_specs=[
            pl.BlockSpec((ROW_BLK, DH), lambda i: (i, 0)),
            pl.BlockSpec((ROW_BLK, DH), lambda i: (i, 0)),
        ],
        out_shape=[
            jax.ShapeDtypeStruct((NP, DH), jnp.float32),
            jax.ShapeDtypeStruct((NP, DH), jnp.float32),
        ],
    )(x, wt, b0, b1)


def _layer_body(a0_ref, a1_ref, dn_ref, b0_ref, b1_ref, wt_ref,
                asv_ref, adv_ref, h0_ref, h1_ref, as_ref, ad_ref):
    dn = dn_ref[...] + jnp.float32(EPS)
    g0 = a0_ref[...] / dn + b0_ref[...]
    g1 = a1_ref[...] / dn + b1_ref[...]
    wt = wt_ref[...]
    h = (jnp.dot(g0, wt[:DH, :], preferred_element_type=jnp.float32)
         + jnp.dot(g1, wt[DH:, :], preferred_element_type=jnp.float32))
    h0_ref[...] = h[:, :DH]
    h1_ref[...] = h[:, DH:]
    as_ref[...] = jnp.sum(h * asv_ref[...], axis=1, keepdims=True)
    ad_ref[...] = jnp.sum(h * adv_ref[...], axis=1, keepdims=True)


def _layer_pre(a0, a1, dn2, b0, b1, wt, asv, adv):
    return pl.pallas_call(
        _layer_body,
        grid=(GRID_R,),
        in_specs=[
            pl.BlockSpec((ROW_BLK, DH), lambda i: (i, 0)),
            pl.BlockSpec((ROW_BLK, DH), lambda i: (i, 0)),
            pl.BlockSpec((ROW_BLK, 1), lambda i: (i, 0)),
            pl.BlockSpec((1, DH), lambda i: (0, 0)),
            pl.BlockSpec((1, DH), lambda i: (0, 0)),
            pl.BlockSpec((D, D), lambda i: (0, 0)),
            pl.BlockSpec((1, D), lambda i: (0, 0)),
            pl.BlockSpec((1, D), lambda i: (0, 0)),
        ],
        out_specs=[
            pl.BlockSpec((ROW_BLK, DH), lambda i: (i, 0)),
            pl.BlockSpec((ROW_BLK, DH), lambda i: (i, 0)),
            pl.BlockSpec((ROW_BLK, 1), lambda i: (i, 0)),
            pl.BlockSpec((ROW_BLK, 1), lambda i: (i, 0)),
        ],
        out_shape=[
            jax.ShapeDtypeStruct((NP, DH), jnp.float32),
            jax.ShapeDtypeStruct((NP, DH), jnp.float32),
            jax.ShapeDtypeStruct((NP, 1), jnp.float32),
            jax.ShapeDtypeStruct((NP, 1), jnp.float32),
        ],
    )(a0, a1, dn2, b0, b1, wt, asv, adv)


def _final_body(s0_ref, s1_ref, dn_ref, b0_ref, b1_ref, wt_ref, bc_ref,
                out_ref):
    dn = dn_ref[...] + jnp.float32(EPS)
    valid = lax.broadcasted_iota(jnp.int32, (NP, 1), 0) < N
    g0 = jnp.where(valid, s0_ref[...] / dn + b0_ref[...], 0.0)
    g1 = jnp.where(valid, s1_ref[...] / dn + b1_ref[...], 0.0)
    m = jnp.concatenate(
        [jnp.sum(g0, axis=0), jnp.sum(g1, axis=0)]) * jnp.float32(1.0 / N)
    out_ref[...] = (jnp.dot(m.reshape(1, D), wt_ref[...],
                            preferred_element_type=jnp.float32)
                    + bc_ref[...])


def _final(s0, s1, dn2, b0, b1, wt, bc):
    return pl.pallas_call(
        _final_body,
        out_shape=jax.ShapeDtypeStruct((1, wt.shape[1]), jnp.float32),
    )(s0, s1, dn2, b0, b1, wt, bc)


@jax.jit
def kernel(x, edge_index, W_in, b_in, Wc, a_src_c, a_dst_c, bc, W_cls, b_cls):
    num_convs = Wc.shape[0]
    loop = jnp.arange(N, dtype=jnp.int32)
    src = jnp.concatenate([edge_index[0].astype(jnp.int32), loop])
    dst = jnp.concatenate([edge_index[1].astype(jnp.int32), loop])
    src = jnp.pad(src, (0, E_PAD - E_TOT))
    dst = jnp.pad(dst, (0, E_PAD - E_TOT))

    xp = jnp.pad(x, ((0, NP - N), (0, 0)))
    z2 = jnp.zeros((NP, DH), jnp.float32)
    z1 = jnp.zeros((NP,), jnp.float32)

    a0, a1 = _proj(xp, W_in.T,
                   b_in[:DH].reshape(1, DH), b_in[DH:].reshape(1, DH))
    dn = jnp.ones((NP,), jnp.float32)
    bprev = jnp.zeros((D,), jnp.float32)
    for i in range(num_convs):
        h0, h1, asv, adv = _layer_pre(
            a0, a1, dn.reshape(NP, 1),
            bprev[:DH].reshape(1, DH), bprev[DH:].reshape(1, DH),
            Wc[i].T, a_src_c[i].reshape(1, D), a_dst_c[i].reshape(1, D))
        exq = _sc_alpha(src, dst, asv.reshape(-1), adv.reshape(-1))
        a0, a1, dn = _sc_edge(src, dst, exq, h0, h1, z2, z1)
        bprev = bc[i]

    return _final(a0, a1, dn.reshape(NP, 1),
                  bprev[:DH].reshape(1, DH), bprev[DH:].reshape(1, DH),
                  W_cls.T, b_cls.reshape(1, -1))


# speedup vs baseline: 1.0151x; 1.0151x over previous
import jax
import jax.numpy as jnp
from jax import lax
from jax.experimental import pallas as pl
from jax.experimental.pallas import tpu as pltpu
from jax.experimental.pallas import tpu_sc as plsc

N = 10000
NP = 10240
D = 256
DH = 128
E_TOT = 330000
CH = 112
NTILES = 16
NCHUNK = 186
E_PAD = NTILES * CH * NCHUNK
EPT = E_PAD // NTILES
RPT = NP // NTILES
EPS = 1e-16
ROW_BLK = 2048
GRID_R = NP // ROW_BLK


def _sc_edge_body(src_hbm, dst_hbm, asrc_hbm, adst_hbm, h0_hbm, h1_hbm,
                  z2_hbm, z1_hbm,
                  s0_out, s1_out, dn_out,
                  sidx0, didx0, ex0, sidx1, didx1, ex1,
                  sidx2, didx2, ex2, sidx3, didx3, ex3,
                  sidx4, didx4, ex4, sidx5, didx5, ex5,
                  rows0, rows1, rows2,
                  av0, bv0, av1, bv1, av2, bv2,
                  s_sh, dn_sh,
                  gsem0, gsem1, gsem2, ssem0, ssem1, ssem2,
                  isem0, isem1, isem2, isem3, isem4, isem5,
                  asem0, asem1, asem2):
    c = lax.axis_index("c")
    s = lax.axis_index("s")
    sidx = [sidx0, sidx1, sidx2, sidx3, sidx4, sidx5]
    didx = [didx0, didx1, didx2, didx3, didx4, didx5]
    exv = [ex0, ex1, ex2, ex3, ex4, ex5]
    rows = [rows0, rows1, rows2]
    av = [av0, av1, av2]
    bv = [bv0, bv1, bv2]
    gsem = [gsem0, gsem1, gsem2]
    ssem = [ssem0, ssem1, ssem2]
    isem = [isem0, isem1, isem2, isem3, isem4, isem5]
    asem = [asem0, asem1, asem2]

    r0 = s * RPT
    pltpu.sync_copy(z2_hbm.at[pl.ds(r0, RPT)], s_sh.at[pl.ds(r0, RPT)])
    pltpu.sync_copy(z1_hbm.at[pl.ds(r0, RPT)], dn_sh.at[pl.ds(r0, RPT)])
    plsc.subcore_barrier()

    tbase = s * EPT

    def idx_load(q, k):
        base = tbase + k * CH
        pltpu.async_copy(src_hbm.at[pl.ds(base, CH)], sidx[q], isem[q])
        pltpu.async_copy(dst_hbm.at[pl.ds(base, CH)], didx[q], isem[q])

    def idx_wait(q, k):
        base = tbase + k * CH
        pltpu.make_async_copy(
            src_hbm.at[pl.ds(base, CH)], sidx[q], isem[q]).wait()
        pltpu.make_async_copy(
            dst_hbm.at[pl.ds(base, CH)], didx[q], isem[q]).wait()

    def gather(q, b):
        @pl.when(c == 0)
        def _():
            pltpu.async_copy(h0_hbm.at[sidx[q]], rows[b], gsem[b])

        @pl.when(c == 1)
        def _():
            pltpu.async_copy(h1_hbm.at[sidx[q]], rows[b], gsem[b])

        pltpu.async_copy(asrc_hbm.at[sidx[q]], av[b], asem[b])
        pltpu.async_copy(adst_hbm.at[didx[q]], bv[b], asem[b])

    def drain_scatter(q, b):
        pltpu.make_async_copy(rows[b], s_sh.at[didx[q]], ssem[b]).wait()
        pltpu.make_async_copy(exv[q], dn_sh.at[didx[q]], ssem[b]).wait()

    def slot(k, u):
        b = u % 3
        q = u % 6
        pltpu.make_async_copy(asrc_hbm.at[sidx[q]], av[b], asem[b]).wait()
        pltpu.make_async_copy(adst_hbm.at[didx[q]], bv[b], asem[b]).wait()
        base = tbase + k * CH

        def wgrp(g, _):
            sl = pl.ds(g * 16, 16)
            al = av[b][sl] + bv[b][sl]
            al = jnp.where(al >= 0.0, al, al * jnp.float32(0.2))
            ex = jnp.exp(al)
            gi = base + g * 16 + lax.iota(jnp.int32, 16)
            exv[q][sl] = jnp.where(gi < E_TOT, ex, jnp.float32(0.0))
            return 0

        lax.fori_loop(0, CH // 16, wgrp, 0)
        pltpu.make_async_copy(h0_hbm.at[sidx[q]], rows[b], gsem[b]).wait()

        def scale(g, _):
            ex16 = exv[q][pl.ds(g * 16, 16)]
            for j in range(16):
                sv = ex16[j]
                row = g * 16 + j
                for r in range(DH // 16):
                    sl = pl.ds(r * 16, 16)
                    rows[b][row, sl] = rows[b][row, sl] * sv
            return 0

        lax.fori_loop(0, CH // 16, scale, 0)
        pltpu.async_copy(rows[b], s_sh.at[didx[q]], ssem[b], add=True)
        pltpu.async_copy(exv[q], dn_sh.at[didx[q]], ssem[b], add=True)

        @pl.when(k >= 1)
        def _():
            drain_scatter((u - 1) % 6, (u - 1) % 3)

        @pl.when(k + 4 < NCHUNK)
        def _():
            idx_load((u + 4) % 6, k + 4)

        @pl.when(k + 2 < NCHUNK)
        def _():
            idx_wait((u + 2) % 6, k + 2)
            gather((u + 2) % 6, (u + 2) % 3)

    for q in range(4):
        idx_load(q, q)
    for j in range(2):
        idx_wait(j, j)
        gather(j, j)

    def group(g, carry):
        for u in range(6):
            slot(6 * g + u, u)
        return carry

    lax.fori_loop(0, NCHUNK // 6, group, 0)
    drain_scatter((NCHUNK - 1) % 6, (NCHUNK - 1) % 3)
    plsc.subcore_barrier()

    @pl.when(c == 0)
    def _():
        pltpu.sync_copy(s_sh.at[pl.ds(r0, RPT)], s0_out.at[pl.ds(r0, RPT)])
        pltpu.sync_copy(dn_sh.at[pl.ds(r0, RPT)], dn_out.at[pl.ds(r0, RPT)])

    @pl.when(c == 1)
    def _():
        pltpu.sync_copy(s_sh.at[pl.ds(r0, RPT)], s1_out.at[pl.ds(r0, RPT)])


_sc_edge = pl.kernel(
    _sc_edge_body,
    out_type=[
        jax.ShapeDtypeStruct((NP, DH), jnp.float32),
        jax.ShapeDtypeStruct((NP, DH), jnp.float32),
        jax.ShapeDtypeStruct((NP,), jnp.float32),
    ],
    mesh=plsc.VectorSubcoreMesh(core_axis_name="c", subcore_axis_name="s"),
    compiler_params=pltpu.CompilerParams(needs_layout_passes=False),
    scratch_types=[
        t for _ in range(6) for t in (
            pltpu.VMEM((CH,), jnp.int32),
            pltpu.VMEM((CH,), jnp.int32),
            pltpu.VMEM((CH,), jnp.float32),
        )
    ] + [pltpu.VMEM((CH, DH), jnp.float32)] * 3
    + [pltpu.VMEM((CH,), jnp.float32)] * 6 + [
        pltpu.VMEM_SHARED((NP, DH), jnp.float32),
        pltpu.VMEM_SHARED((NP,), jnp.float32),
    ] + [pltpu.SemaphoreType.DMA] * 15,
)


def _proj_body(x_ref, wt_ref, b0_ref, b1_ref, o0_ref, o1_ref):
    h = jnp.dot(x_ref[...], wt_ref[...], preferred_element_type=jnp.float32)
    o0_ref[...] = h[:, :DH] + b0_ref[...]
    o1_ref[...] = h[:, DH:] + b1_ref[...]


def _proj(x, wt, b0, b1):
    return pl.pallas_call(
        _proj_body,
        grid=(GRID_R,),
        in_specs=[
            pl.BlockSpec((ROW_BLK, x.shape[1]), lambda i: (i, 0)),
            pl.BlockSpec(wt.shape, lambda i: (0, 0)),
            pl.BlockSpec((1, DH), lambda i: (0, 0)),
            pl.BlockSpec((1, DH), lambda i: (0, 0)),
        ],
        out_specs=[
            pl.BlockSpec((ROW_BLK, DH), lambda i: (i, 0)),
            pl.BlockSpec((ROW_BLK, DH), lambda i: (i, 0)),
        ],
        out_shape=[
            jax.ShapeDtypeStruct((NP, DH), jnp.float32),
            jax.ShapeDtypeStruct((NP, DH), jnp.float32),
        ],
    )(x, wt, b0, b1)


def _layer_body(a0_ref, a1_ref, dn_ref, b0_ref, b1_ref, wt_ref,
                asv_ref, adv_ref, h0_ref, h1_ref, as_ref, ad_ref):
    dn = dn_ref[...] + jnp.float32(EPS)
    g0 = a0_ref[...] / dn + b0_ref[...]
    g1 = a1_ref[...] / dn + b1_ref[...]
    wt = wt_ref[...]
    h = (jnp.dot(g0, wt[:DH, :], preferred_element_type=jnp.float32)
         + jnp.dot(g1, wt[DH:, :], preferred_element_type=jnp.float32))
    h0_ref[...] = h[:, :DH]
    h1_ref[...] = h[:, DH:]
    as_ref[...] = jnp.sum(h * asv_ref[...], axis=1, keepdims=True)
    ad_ref[...] = jnp.sum(h * adv_ref[...], axis=1, keepdims=True)


def _layer_pre(a0, a1, dn2, b0, b1, wt, asv, adv):
    return pl.pallas_call(
        _layer_body,
        grid=(GRID_R,),
        in_specs=[
            pl.BlockSpec((ROW_BLK, DH), lambda i: (i, 0)),
            pl.BlockSpec((ROW_BLK, DH), lambda i: (i, 0)),
            pl.BlockSpec((ROW_BLK, 1), lambda i: (i, 0)),
            pl.BlockSpec((1, DH), lambda i: (0, 0)),
            pl.BlockSpec((1, DH), lambda i: (0, 0)),
            pl.BlockSpec((D, D), lambda i: (0, 0)),
            pl.BlockSpec((1, D), lambda i: (0, 0)),
            pl.BlockSpec((1, D), lambda i: (0, 0)),
        ],
        out_specs=[
            pl.BlockSpec((ROW_BLK, DH), lambda i: (i, 0)),
            pl.BlockSpec((ROW_BLK, DH), lambda i: (i, 0)),
            pl.BlockSpec((ROW_BLK, 1), lambda i: (i, 0)),
            pl.BlockSpec((ROW_BLK, 1), lambda i: (i, 0)),
        ],
        out_shape=[
            jax.ShapeDtypeStruct((NP, DH), jnp.float32),
            jax.ShapeDtypeStruct((NP, DH), jnp.float32),
            jax.ShapeDtypeStruct((NP, 1), jnp.float32),
            jax.ShapeDtypeStruct((NP, 1), jnp.float32),
        ],
    )(a0, a1, dn2, b0, b1, wt, asv, adv)


def _final_body(s0_ref, s1_ref, dn_ref, b0_ref, b1_ref, wt_ref, bc_ref,
                out_ref):
    dn = dn_ref[...] + jnp.float32(EPS)
    valid = lax.broadcasted_iota(jnp.int32, (NP, 1), 0) < N
    g0 = jnp.where(valid, s0_ref[...] / dn + b0_ref[...], 0.0)
    g1 = jnp.where(valid, s1_ref[...] / dn + b1_ref[...], 0.0)
    m = jnp.concatenate(
        [jnp.sum(g0, axis=0), jnp.sum(g1, axis=0)]) * jnp.float32(1.0 / N)
    out_ref[...] = (jnp.dot(m.reshape(1, D), wt_ref[...],
                            preferred_element_type=jnp.float32)
                    + bc_ref[...])


def _final(s0, s1, dn2, b0, b1, wt, bc):
    return pl.pallas_call(
        _final_body,
        out_shape=jax.ShapeDtypeStruct((1, wt.shape[1]), jnp.float32),
    )(s0, s1, dn2, b0, b1, wt, bc)


@jax.jit
def kernel(x, edge_index, W_in, b_in, Wc, a_src_c, a_dst_c, bc, W_cls, b_cls):
    num_convs = Wc.shape[0]
    loop = jnp.arange(N, dtype=jnp.int32)
    src = jnp.concatenate([edge_index[0].astype(jnp.int32), loop])
    dst = jnp.concatenate([edge_index[1].astype(jnp.int32), loop])
    src = jnp.pad(src, (0, E_PAD - E_TOT))
    dst = jnp.pad(dst, (0, E_PAD - E_TOT))

    xp = jnp.pad(x, ((0, NP - N), (0, 0)))
    z2 = jnp.zeros((NP, DH), jnp.float32)
    z1 = jnp.zeros((NP,), jnp.float32)

    a0, a1 = _proj(xp, W_in.T,
                   b_in[:DH].reshape(1, DH), b_in[DH:].reshape(1, DH))
    dn = jnp.ones((NP,), jnp.float32)
    bprev = jnp.zeros((D,), jnp.float32)
    for i in range(num_convs):
        h0, h1, asv, adv = _layer_pre(
            a0, a1, dn.reshape(NP, 1),
            bprev[:DH].reshape(1, DH), bprev[DH:].reshape(1, DH),
            Wc[i].T, a_src_c[i].reshape(1, D), a_dst_c[i].reshape(1, D))
        a0, a1, dn = _sc_edge(src, dst, asv.reshape(-1), adv.reshape(-1),
                              h0, h1, z2, z1)
        bprev = bc[i]

    return _final(a0, a1, dn.reshape(NP, 1),
                  bprev[:DH].reshape(1, DH), bprev[DH:].reshape(1, DH),
                  W_cls.T, b_cls.reshape(1, -1))
